# baseline (speedup 1.0000x reference)
import functools

import jax
import jax.numpy as jnp
from jax import lax
from jax.experimental import pallas as pl
from jax.experimental.pallas import tpu as pltpu
from jax.experimental.pallas import tpu_sc as plsc

_V = 1000
_S = 8
_P = 128
_B = 1024
_T = 200
_N = _B * _T

_NC = 2
_NS = 16
_NW = _NC * _NS
_L = 16
_ROWS_PER_W = _N // _NW
_WROWS = 128
_NWIN = _ROWS_PER_W // _WROWS
_NSTEP = _S * _NWIN


def _lse_body(table_ref, lse_ref):
    x = table_ref[...]
    m = jnp.max(x, axis=1, keepdims=True)
    s = jnp.sum(jnp.exp(x - m), axis=1, keepdims=True)
    lse_ref[...] = m + jnp.log(s)


def _compute_lse(table):
    return pl.pallas_call(
        _lse_body,
        out_shape=jax.ShapeDtypeStruct((_V, 1), jnp.float32),
    )(table)


_FB = 64


def _fmt_body(*refs):
    *ins, out_ref = refs
    for c in range(_S - 1):
        out_ref[:, c * _P:(c + 1) * _P] = ins[c][...]
    out_ref[:, (_S - 1) * _P:_V] = ins[_S - 1][:, :_V - (_S - 1) * _P]


def _format(out8c):
    return pl.pallas_call(
        _fmt_body,
        grid=(_N // _FB,),
        in_specs=[
            pl.BlockSpec((_FB, _P), functools.partial(
                lambda c, i: (c * (_N // _FB) + i, 0), c))
            for c in range(_S)
        ],
        out_specs=pl.BlockSpec((_FB, _V), lambda i: (i, 0)),
        out_shape=jax.ShapeDtypeStruct((_N, _V), jnp.float32),
    )(*([out8c] * _S))


def _sc_body(table_hbm, idx_hbm, tgt_hbm, lse_hbm, out_hbm, part_hbm,
             idx_v, tgt_v, ib0_v, ib1_v, rows0_v, rows1_v, lse_v, acc_v,
             sg0, sg1, so0, so1):
    wid = lax.axis_index("s") * _NC + lax.axis_index("c")
    base = wid * _ROWS_PER_W
    rows = (rows0_v, rows1_v)
    ib = (ib0_v, ib1_v)
    sg = (sg0, sg1)
    so = (so0, so1)

    pltpu.sync_copy(lse_hbm, lse_v)
    pltpu.sync_copy(idx_hbm.at[pl.ds(base, _ROWS_PER_W)], idx_v)
    pltpu.sync_copy(tgt_hbm.at[pl.ds(base, _ROWS_PER_W)], tgt_v)
    acc_v[...] = jnp.zeros((_L,), jnp.float32)

    lanes = lax.iota(jnp.int32, _L)

    def build_and_gather(g, b):
        c = g // _NWIN
        w = g - c * _NWIN
        off = w * _WROWS
        for j in range(_WROWS // _L):
            iv = idx_v[pl.ds(off + j * _L, _L)]
            ib[b][pl.ds(j * _L, _L)] = iv * _S + c
        pltpu.async_copy(table_hbm.at[ib[b]], rows[b], sg[b])

    def picks(g, b):
        c = g // _NWIN
        w = g - c * _NWIN
        off = w * _WROWS
        for j in range(_WROWS // _L):
            tv = tgt_v[pl.ds(off + j * _L, _L)]
            qi = tv >> 7
            rm = tv & 127
            mask = qi == c
            nloc = lanes + j * _L
            picked = plsc.load_gather(rows[b], [nloc, rm], mask=mask)
            acc_v[...] = acc_v[...] - jnp.where(mask, picked, 0.0)
            @pl.when(c == 0)
            def _():
                iv = idx_v[pl.ds(off + j * _L, _L)]
                acc_v[...] = acc_v[...] + plsc.load_gather(lse_v, [iv])

    def out_off(g):
        c = g // _NWIN
        w = g - c * _NWIN
        return c * _N + base + w * _WROWS

    build_and_gather(0, 0)
    build_and_gather(1, 1)

    def step(h, carry):
        for b in range(2):
            g = h * 2 + b
            pltpu.make_async_copy(
                table_hbm.at[ib[b]], rows[b], sg[b]
            ).wait()
            pltpu.async_copy(rows[b], out_hbm.at[pl.ds(out_off(g), _WROWS)],
                             so[b])
            picks(g, b)
            @pl.when(g + 2 < _NSTEP)
            def _():
                pltpu.make_async_copy(
                    rows[b], out_hbm.at[pl.ds(0, _WROWS)], so[b]
                ).wait()
                build_and_gather(g + 2, b)
        return carry

    lax.fori_loop(0, _NSTEP // 2, step, 0)
    for b in range(2):
        pltpu.make_async_copy(
            rows[b], out_hbm.at[pl.ds(0, _WROWS)], so[b]
        ).wait()
    pltpu.sync_copy(acc_v, part_hbm.at[pl.ds(wid * _L, _L)])


@functools.partial(
    pl.kernel,
    out_type=[
        jax.ShapeDtypeStruct((_S * _N, _P), jnp.float32),
        jax.ShapeDtypeStruct((_NW * _L,), jnp.float32),
    ],
    mesh=plsc.VectorSubcoreMesh(core_axis_name="c", subcore_axis_name="s",
                                num_cores=_NC, num_subcores=_NS),
    compiler_params=pltpu.CompilerParams(needs_layout_passes=False,
                                         use_tc_tiling_on_sc=False),
    scratch_types=[
        pltpu.VMEM((_ROWS_PER_W,), jnp.int32),
        pltpu.VMEM((_ROWS_PER_W,), jnp.int32),
        pltpu.VMEM((_WROWS,), jnp.int32),
        pltpu.VMEM((_WROWS,), jnp.int32),
        pltpu.VMEM((_WROWS, _P), jnp.float32),
        pltpu.VMEM((_WROWS, _P), jnp.float32),
        pltpu.VMEM((_V,), jnp.float32),
        pltpu.VMEM((_L,), jnp.float32),
        pltpu.SemaphoreType.DMA,
        pltpu.SemaphoreType.DMA,
        pltpu.SemaphoreType.DMA,
        pltpu.SemaphoreType.DMA,
    ],
)
def _sc_kernel(table_hbm, idx_hbm, tgt_hbm, lse_hbm, out_hbm, part_hbm,
               idx_v, tgt_v, ib0_v, ib1_v, rows0_v, rows1_v, lse_v, acc_v,
               sg0, sg1, so0, so1):
    _sc_body(table_hbm, idx_hbm, tgt_hbm, lse_hbm, out_hbm, part_hbm,
             idx_v, tgt_v, ib0_v, ib1_v, rows0_v, rows1_v, lse_v, acc_v,
             sg0, sg1, so0, so1)


def kernel(idx, targets, table):
    idx_f = idx.reshape(_N).astype(jnp.int32)
    tgt_f = targets.reshape(_N).astype(jnp.int32)
    table8 = jnp.pad(table, ((0, 0), (0, _S * _P - _V))).reshape(_V * _S, _P)
    lse = _compute_lse(table).reshape(_V)
    out8c, part = _sc_kernel(table8, idx_f, tgt_f, lse)
    logits = _format(out8c)
    loss = jnp.sum(part) / _N
    return (logits, loss)


# speedup vs baseline: 1.4570x; 1.4570x over previous
import functools

import jax
import jax.numpy as jnp
from jax import lax
from jax.experimental import pallas as pl
from jax.experimental.pallas import tpu as pltpu
from jax.experimental.pallas import tpu_sc as plsc

_V = 1000
_S = 8
_P = 128
_B = 1024
_T = 200
_N = _B * _T

_NC = 2
_NS = 16
_NW = _NC * _NS
_L = 16
_ROWS_PER_W = _N // _NW
_WROWS = 128
_NWIN = _ROWS_PER_W // _WROWS
_NSTEP = _S * _NWIN


def _lse_body(table_ref, lse_ref):
    x = table_ref[...]
    m = jnp.max(x, axis=1, keepdims=True)
    s = jnp.sum(jnp.exp(x - m), axis=1, keepdims=True)
    lse_ref[...] = m + jnp.log(s)


def _compute_lse(table):
    return pl.pallas_call(
        _lse_body,
        out_shape=jax.ShapeDtypeStruct((_V, 1), jnp.float32),
    )(table)


_FB = 256


def _fmt_body(in_ref, out_ref):
    x = in_ref[...]
    for c in range(_S - 1):
        out_ref[:, c * _P:(c + 1) * _P] = x[c]
    out_ref[:, (_S - 1) * _P:_V] = x[_S - 1][:, :_V - (_S - 1) * _P]


def _format(out8c):
    return pl.pallas_call(
        _fmt_body,
        grid=(_N // _FB,),
        in_specs=[pl.BlockSpec((_S, _FB, _P), lambda i: (0, i, 0))],
        out_specs=pl.BlockSpec((_FB, _V), lambda i: (i, 0)),
        out_shape=jax.ShapeDtypeStruct((_N, _V), jnp.float32),
    )(out8c.reshape(_S, _N, _P))


def _sc_body(table_hbm, idx_hbm, tgt_hbm, lse_hbm, out_hbm, part_hbm,
             idx_v, tgt_v, ib0_v, ib1_v, rows0_v, rows1_v, lse_v, acc_v,
             sg0, sg1, so0, so1):
    wid = lax.axis_index("s") * _NC + lax.axis_index("c")
    base = wid * _ROWS_PER_W
    rows = (rows0_v, rows1_v)
    ib = (ib0_v, ib1_v)
    sg = (sg0, sg1)
    so = (so0, so1)

    pltpu.sync_copy(lse_hbm, lse_v)
    pltpu.sync_copy(idx_hbm.at[pl.ds(base, _ROWS_PER_W)], idx_v)
    pltpu.sync_copy(tgt_hbm.at[pl.ds(base, _ROWS_PER_W)], tgt_v)
    acc_v[...] = jnp.zeros((_L,), jnp.float32)

    lanes = lax.iota(jnp.int32, _L)

    def build_and_gather(g, b):
        c = g // _NWIN
        w = g - c * _NWIN
        off = w * _WROWS
        for j in range(_WROWS // _L):
            iv = idx_v[pl.ds(off + j * _L, _L)]
            ib[b][pl.ds(j * _L, _L)] = iv * _S + c
        pltpu.async_copy(table_hbm.at[ib[b]], rows[b], sg[b])

    def picks(g, b):
        c = g // _NWIN
        w = g - c * _NWIN
        off = w * _WROWS
        for j in range(_WROWS // _L):
            tv = tgt_v[pl.ds(off + j * _L, _L)]
            qi = tv >> 7
            rm = tv & 127
            mask = qi == c
            nloc = lanes + j * _L
            picked = plsc.load_gather(rows[b], [nloc, rm], mask=mask)
            acc_v[...] = acc_v[...] - jnp.where(mask, picked, 0.0)
            @pl.when(c == 0)
            def _():
                iv = idx_v[pl.ds(off + j * _L, _L)]
                acc_v[...] = acc_v[...] + plsc.load_gather(lse_v, [iv])

    def out_off(g):
        c = g // _NWIN
        w = g - c * _NWIN
        return c * _N + base + w * _WROWS

    build_and_gather(0, 0)
    build_and_gather(1, 1)

    def step(h, carry):
        for b in range(2):
            g = h * 2 + b
            pltpu.make_async_copy(
                table_hbm.at[ib[b]], rows[b], sg[b]
            ).wait()
            pltpu.async_copy(rows[b], out_hbm.at[pl.ds(out_off(g), _WROWS)],
                             so[b])
            picks(g, b)
            @pl.when(g + 2 < _NSTEP)
            def _():
                pltpu.make_async_copy(
                    rows[b], out_hbm.at[pl.ds(0, _WROWS)], so[b]
                ).wait()
                build_and_gather(g + 2, b)
        return carry

    lax.fori_loop(0, _NSTEP // 2, step, 0)
    for b in range(2):
        pltpu.make_async_copy(
            rows[b], out_hbm.at[pl.ds(0, _WROWS)], so[b]
        ).wait()
    pltpu.sync_copy(acc_v, part_hbm.at[pl.ds(wid * _L, _L)])


@functools.partial(
    pl.kernel,
    out_type=[
        jax.ShapeDtypeStruct((_S * _N, _P), jnp.float32),
        jax.ShapeDtypeStruct((_NW * _L,), jnp.float32),
    ],
    mesh=plsc.VectorSubcoreMesh(core_axis_name="c", subcore_axis_name="s",
                                num_cores=_NC, num_subcores=_NS),
    compiler_params=pltpu.CompilerParams(needs_layout_passes=False,
                                         use_tc_tiling_on_sc=False),
    scratch_types=[
        pltpu.VMEM((_ROWS_PER_W,), jnp.int32),
        pltpu.VMEM((_ROWS_PER_W,), jnp.int32),
        pltpu.VMEM((_WROWS,), jnp.int32),
        pltpu.VMEM((_WROWS,), jnp.int32),
        pltpu.VMEM((_WROWS, _P), jnp.float32),
        pltpu.VMEM((_WROWS, _P), jnp.float32),
        pltpu.VMEM((_V,), jnp.float32),
        pltpu.VMEM((_L,), jnp.float32),
        pltpu.SemaphoreType.DMA,
        pltpu.SemaphoreType.DMA,
        pltpu.SemaphoreType.DMA,
        pltpu.SemaphoreType.DMA,
    ],
)
def _sc_kernel(table_hbm, idx_hbm, tgt_hbm, lse_hbm, out_hbm, part_hbm,
               idx_v, tgt_v, ib0_v, ib1_v, rows0_v, rows1_v, lse_v, acc_v,
               sg0, sg1, so0, so1):
    _sc_body(table_hbm, idx_hbm, tgt_hbm, lse_hbm, out_hbm, part_hbm,
             idx_v, tgt_v, ib0_v, ib1_v, rows0_v, rows1_v, lse_v, acc_v,
             sg0, sg1, so0, so1)


def kernel(idx, targets, table):
    idx_f = idx.reshape(_N).astype(jnp.int32)
    tgt_f = targets.reshape(_N).astype(jnp.int32)
    table8 = jnp.pad(table, ((0, 0), (0, _S * _P - _V))).reshape(_V * _S, _P)
    lse = _compute_lse(table).reshape(_V)
    out8c, part = _sc_kernel(table8, idx_f, tgt_f, lse)
    logits = _format(out8c)
    loss = jnp.sum(part) / _N
    return (logits, loss)


# speedup vs baseline: 1.7510x; 1.2018x over previous
import functools

import jax
import jax.numpy as jnp
from jax import lax
from jax.experimental import pallas as pl
from jax.experimental.pallas import tpu as pltpu
from jax.experimental.pallas import tpu_sc as plsc

_V = 1000
_B = 1024
_T = 200
_N = _B * _T

_NC = 2
_NS = 16
_NW = _NC * _NS
_L = 16
_ROWS_PER_W = _N // _NW
_CHUNK = 32
_NCHUNK = _ROWS_PER_W // _CHUNK


def _lse_body(table_ref, lse_ref):
    x = table_ref[...]
    m = jnp.max(x, axis=1, keepdims=True)
    s = jnp.sum(jnp.exp(x - m), axis=1, keepdims=True)
    lse_ref[...] = m + jnp.log(s)


def _compute_lse(table):
    return pl.pallas_call(
        _lse_body,
        out_shape=jax.ShapeDtypeStruct((_V, 1), jnp.float32),
    )(table)


def _sc_body(table_hbm, idx_hbm, tgt_hbm, lse_hbm, out_hbm, part_hbm,
             idx_v, tgt_v, rows0_v, rows1_v, lse_v, acc_v,
             sg0, sg1, so0, so1):
    wid = lax.axis_index("s") * _NC + lax.axis_index("c")
    base = wid * _ROWS_PER_W
    rows = (rows0_v, rows1_v)
    sg = (sg0, sg1)
    so = (so0, so1)

    pltpu.sync_copy(lse_hbm, lse_v)
    pltpu.sync_copy(idx_hbm.at[pl.ds(base, _ROWS_PER_W)], idx_v)
    pltpu.sync_copy(tgt_hbm.at[pl.ds(base, _ROWS_PER_W)], tgt_v)
    acc_v[...] = jnp.zeros((_L,), jnp.float32)

    def gather_chunk(g, b):
        pltpu.async_copy(table_hbm.at[idx_v.at[pl.ds(g * _CHUNK, _CHUNK)]],
                         rows[b], sg[b])

    def picks(g, b):
        off = g * _CHUNK
        for j in range(_CHUNK // _L):
            iv = idx_v[pl.ds(off + j * _L, _L)]
            tv = tgt_v[pl.ds(off + j * _L, _L)]
            lse_g = plsc.load_gather(lse_v, [iv])
            nloc = lax.iota(jnp.int32, _L) + j * _L
            picked = plsc.load_gather(rows[b], [nloc, tv])
            acc_v[...] = acc_v[...] + (lse_g - picked)

    gather_chunk(0, 0)
    gather_chunk(1, 1)

    def step(h, carry):
        for b in range(2):
            g = h * 2 + b
            pltpu.make_async_copy(
                table_hbm.at[idx_v.at[pl.ds(0, _CHUNK)]], rows[b], sg[b]
            ).wait()
            pltpu.async_copy(rows[b],
                             out_hbm.at[pl.ds(base + g * _CHUNK, _CHUNK)],
                             so[b])
            picks(g, b)
            @pl.when(g + 2 < _NCHUNK)
            def _():
                pltpu.make_async_copy(
                    rows[b], out_hbm.at[pl.ds(0, _CHUNK)], so[b]
                ).wait()
                gather_chunk(g + 2, b)
        return carry

    lax.fori_loop(0, _NCHUNK // 2, step, 0)
    for b in range(2):
        pltpu.make_async_copy(
            rows[b], out_hbm.at[pl.ds(0, _CHUNK)], so[b]
        ).wait()
    pltpu.sync_copy(acc_v, part_hbm.at[pl.ds(wid * _L, _L)])


@functools.partial(
    pl.kernel,
    out_type=[
        jax.ShapeDtypeStruct((_N, _V), jnp.float32),
        jax.ShapeDtypeStruct((_NW * _L,), jnp.float32),
    ],
    mesh=plsc.VectorSubcoreMesh(core_axis_name="c", subcore_axis_name="s",
                                num_cores=_NC, num_subcores=_NS),
    compiler_params=pltpu.CompilerParams(needs_layout_passes=False,
                                         use_tc_tiling_on_sc=False),
    scratch_types=[
        pltpu.VMEM((_ROWS_PER_W,), jnp.int32),
        pltpu.VMEM((_ROWS_PER_W,), jnp.int32),
        pltpu.VMEM((_CHUNK, _V), jnp.float32),
        pltpu.VMEM((_CHUNK, _V), jnp.float32),
        pltpu.VMEM((_V,), jnp.float32),
        pltpu.VMEM((_L,), jnp.float32),
        pltpu.SemaphoreType.DMA,
        pltpu.SemaphoreType.DMA,
        pltpu.SemaphoreType.DMA,
        pltpu.SemaphoreType.DMA,
    ],
)
def _sc_kernel(table_hbm, idx_hbm, tgt_hbm, lse_hbm, out_hbm, part_hbm,
               idx_v, tgt_v, rows0_v, rows1_v, lse_v, acc_v,
               sg0, sg1, so0, so1):
    _sc_body(table_hbm, idx_hbm, tgt_hbm, lse_hbm, out_hbm, part_hbm,
             idx_v, tgt_v, rows0_v, rows1_v, lse_v, acc_v,
             sg0, sg1, so0, so1)


def kernel(idx, targets, table):
    idx_f = idx.reshape(_N).astype(jnp.int32)
    tgt_f = targets.reshape(_N).astype(jnp.int32)
    lse = _compute_lse(table).reshape(_V)
    logits, part = _sc_kernel(table, idx_f, tgt_f, lse)
    loss = jnp.sum(part) / _N
    return (logits, loss)
